# baseline (speedup 1.0000x reference)
import jax
import jax.numpy as jnp
from jax import lax
from jax.experimental import pallas as pl
from jax.experimental.pallas import tpu as pltpu
from jax.experimental.pallas import tpu_sc as plsc


def _pad_transpose(tT):
    Dd, V = tT.shape
    BLK = 512

    def body(t_ref, o_ref):
        o_ref[:, pl.ds(0, Dd)] = t_ref[...].T
        o_ref[:, pl.ds(Dd, 128 - Dd)] = jnp.zeros((BLK, 128 - Dd), jnp.float32)

    return pl.pallas_call(
        body,
        grid=(pl.cdiv(V, BLK),),
        in_specs=[pl.BlockSpec((Dd, BLK), lambda i: (0, i))],
        out_specs=pl.BlockSpec((BLK, 128), lambda i: (i, 0)),
        out_shape=jax.ShapeDtypeStruct((V, 128), jnp.float32),
    )(tT)


def kernel(x, seq_lengths, table):
    del seq_lengths
    B, L = x.shape
    V, D = table.shape
    NB = B // 128
    DG = D // 8
    xT = x.T.astype(jnp.int32) * 2
    tp = _pad_transpose(table.T).reshape(2 * V, D)

    mesh = plsc.VectorSubcoreMesh(core_axis_name="core", subcore_axis_name="subcore")

    @pl.kernel(
        out_type=jax.ShapeDtypeStruct((L, DG, NB, 8, 128), table.dtype),
        mesh=mesh,
        compiler_params=pltpu.CompilerParams(
            use_tc_tiling_on_sc=False, needs_layout_passes=False
        ),
        scratch_types=[
            pltpu.VMEM((L, 128), jnp.int32),
            pltpu.VMEM((128, D), jnp.float32),
            pltpu.VMEM((128, D), jnp.float32),
            pltpu.VMEM((DG, 8, 129), jnp.float32),
            pltpu.VMEM((DG, 8, 129), jnp.float32),
            pltpu.SemaphoreType.DMA,
            pltpu.SemaphoreType.DMA,
            pltpu.SemaphoreType.DMA,
            pltpu.SemaphoreType.DMA,
        ],
    )
    def gather_kernel(table_hbm, xT_hbm, z_hbm, idx_v, g0, g1, o0, o1, sg0, sg1, so0, so1):
        wid = lax.axis_index("subcore") * 2 + lax.axis_index("core")
        bt = wid

        pltpu.sync_copy(xT_hbm.at[:, pl.ds(bt * 128, 128)], idx_v)

        def start_gather(l, g, sem):
            pltpu.async_copy(table_hbm.at[idx_v.at[l]], g, sem)

        iota = lax.iota(jnp.int32, 16)
        dgvs = [(iota + 16 * c) // 8 for c in range(4)]
        divs = [(iota + 16 * c) % 8 for c in range(4)]

        def transpose(g, o):
            @pl.loop(0, 128, step=2)
            def _(b):
                for u in range(2):
                    bv = jnp.full((16,), 1, jnp.int32) * (b + u)
                    for c in range(4):
                        vals = g[b + u, pl.ds(c * 16, 16)]
                        plsc.store_scatter(o, [dgvs[c], divs[c], bv], vals)

        def flush(o, so, l):
            pltpu.async_copy(o.at[:, :, pl.ds(0, 128)], z_hbm.at[l, :, bt], so)

        def wait_gather(g, sem):
            pltpu.make_async_copy(table_hbm.at[idx_v.at[0]], g, sem).wait()

        def wait_out(o, so, l):
            pltpu.make_async_copy(
                o.at[:, :, pl.ds(0, 128)], z_hbm.at[l, :, bt], so
            ).wait()

        start_gather(0, g0, sg0)

        @pl.loop(0, L // 2)
        def _(i):
            l0 = i * 2
            l1 = l0 + 1
            start_gather(l1, g1, sg1)
            wait_gather(g0, sg0)

            @pl.when(i > 0)
            def _():
                wait_out(o0, so0, l0)

            transpose(g0, o0)
            flush(o0, so0, l0)

            @pl.when(i < L // 2 - 1)
            def _():
                start_gather(l0 + 2, g0, sg0)

            wait_gather(g1, sg1)

            @pl.when(i > 0)
            def _():
                wait_out(o1, so1, l1)

            transpose(g1, o1)
            flush(o1, so1, l1)

        wait_out(o0, so0, 0)
        wait_out(o1, so1, 0)

    z = gather_kernel(tp, xT)
    return z.transpose(2, 4, 0, 1, 3).reshape(B, L, D)


# speedup vs baseline: 1.4030x; 1.4030x over previous
import jax
import jax.numpy as jnp
from jax import lax
from jax.experimental import pallas as pl
from jax.experimental.pallas import tpu as pltpu
from jax.experimental.pallas import tpu_sc as plsc


def kernel(x, seq_lengths, table):
    del seq_lengths
    B, L = x.shape
    V, D = table.shape
    NB = B // 128
    DG = D // 8
    xT = x.T.astype(jnp.int32) * 2
    tp = jnp.pad(table, ((0, 0), (0, 128 - D))).reshape(2 * V, D)

    mesh = plsc.VectorSubcoreMesh(core_axis_name="core", subcore_axis_name="subcore")

    @pl.kernel(
        out_type=jax.ShapeDtypeStruct((L, DG, NB, 8, 128), table.dtype),
        mesh=mesh,
        compiler_params=pltpu.CompilerParams(
            use_tc_tiling_on_sc=False, needs_layout_passes=False
        ),
        scratch_types=[
            pltpu.VMEM((L, 128), jnp.int32),
            pltpu.VMEM((128, D), jnp.float32),
            pltpu.VMEM((128, D), jnp.float32),
            pltpu.VMEM((DG, 8, 129), jnp.float32),
            pltpu.VMEM((DG, 8, 129), jnp.float32),
            pltpu.SemaphoreType.DMA,
            pltpu.SemaphoreType.DMA,
            pltpu.SemaphoreType.DMA,
            pltpu.SemaphoreType.DMA,
        ],
    )
    def gather_kernel(table_hbm, xT_hbm, z_hbm, idx_v, g0, g1, o0, o1, sg0, sg1, so0, so1):
        wid = lax.axis_index("subcore") * 2 + lax.axis_index("core")
        bt = wid

        pltpu.sync_copy(xT_hbm.at[:, pl.ds(bt * 128, 128)], idx_v)

        def start_gather(l, g, sem):
            pltpu.async_copy(table_hbm.at[idx_v.at[l]], g, sem)

        iota = lax.iota(jnp.int32, 16)
        dgvs = [(iota + 16 * c) // 8 for c in range(4)]
        divs = [(iota + 16 * c) % 8 for c in range(4)]

        def transpose(g, o):
            @pl.loop(0, 128, step=2)
            def _(b):
                for u in range(2):
                    bv = jnp.full((16,), 1, jnp.int32) * (b + u)
                    for c in range(4):
                        vals = g[b + u, pl.ds(c * 16, 16)]
                        plsc.store_scatter(o, [dgvs[c], divs[c], bv], vals)

        def flush(o, so, l):
            pltpu.async_copy(o.at[:, :, pl.ds(0, 128)], z_hbm.at[l, :, bt], so)

        def wait_gather(g, sem):
            pltpu.make_async_copy(table_hbm.at[idx_v.at[0]], g, sem).wait()

        def wait_out(o, so, l):
            pltpu.make_async_copy(
                o.at[:, :, pl.ds(0, 128)], z_hbm.at[l, :, bt], so
            ).wait()

        start_gather(0, g0, sg0)

        @pl.loop(0, L // 2)
        def _(i):
            l0 = i * 2
            l1 = l0 + 1
            start_gather(l1, g1, sg1)
            wait_gather(g0, sg0)

            @pl.when(i > 0)
            def _():
                wait_out(o0, so0, l0)

            transpose(g0, o0)
            flush(o0, so0, l0)

            @pl.when(i < L // 2 - 1)
            def _():
                start_gather(l0 + 2, g0, sg0)

            wait_gather(g1, sg1)

            @pl.when(i > 0)
            def _():
                wait_out(o1, so1, l1)

            transpose(g1, o1)
            flush(o1, so1, l1)

        wait_out(o0, so0, 0)
        wait_out(o1, so1, 0)

    z = gather_kernel(tp, xT)
    return z.transpose(2, 4, 0, 1, 3).reshape(B, L, D)
